# Initial kernel scaffold; baseline (speedup 1.0000x reference)
import jax
import jax.numpy as jnp
from jax.experimental import pallas as pl


def kernel(hidden_states, pe_k_weight):
    raise NotImplementedError("write your pallas kernel here")


# speedup vs baseline: 2.1270x; 2.1270x over previous
import jax
import jax.numpy as jnp
from jax import lax
from jax.experimental import pallas as pl
from jax.experimental.pallas import tpu as pltpu

_DIM = 768
_ML = 1000
_S = 512
_BJ = 8
_NST = _S // _BJ


def _body(r_hbm, out_hbm, w, tsh, load_sem, sems):
    jb = pl.program_id(0)
    p = jb % 2

    @pl.when(jb == 0)
    def _stage():
        cp = pltpu.make_async_copy(r_hbm.at[pl.ds(488, 1032), :], w, load_sem)
        cp.start()
        cp.wait()

        def _chunk(k, carry):
            base = pl.multiple_of(8 * k, 8)
            v = w[pl.ds(base, 16), :]
            for c in range(8):
                tsh[c, pl.ds(base, 8), :] = v[c:c + 8, :]
            return carry

        lax.fori_loop(0, 1024 // 8, _chunk, 0)

    def _copy(d, slot):
        j = jb * _BJ + d
        c = (8 - d) % 8
        off = 8 * (64 - jb) if d == 0 else 8 * (63 - jb)
        off = pl.multiple_of(off, 8)
        return pltpu.make_async_copy(
            tsh.at[c, pl.ds(off, _S), :],
            out_hbm.at[:, pl.ds(j * _DIM, _DIM)],
            sems.at[slot, d],
        )

    for d in range(_BJ):
        _copy(d, p).start()

    @pl.when(jb > 0)
    def _drain_prev():
        for d in range(_BJ):
            _copy(d, 1 - p).wait()

    @pl.when(jb == _NST - 1)
    def _drain_last():
        for d in range(_BJ):
            _copy(d, p).wait()


def kernel(hidden_states, pe_k_weight):
    s = hidden_states.shape[1]
    out2d = pl.pallas_call(
        _body,
        grid=(_NST,),
        in_specs=[pl.BlockSpec(memory_space=pl.ANY)],
        out_specs=pl.BlockSpec(memory_space=pl.ANY),
        out_shape=jax.ShapeDtypeStruct((s, s * _DIM), jnp.float32),
        scratch_shapes=[
            pltpu.VMEM((1032, _DIM), jnp.float32),
            pltpu.VMEM((8, 1024, _DIM), jnp.float32),
            pltpu.SemaphoreType.DMA,
            pltpu.SemaphoreType.DMA((2, _BJ)),
        ],
    )(pe_k_weight)
    return out2d.reshape(s, s, _DIM)
